# Initial kernel scaffold; baseline (speedup 1.0000x reference)
import jax
import jax.numpy as jnp
from jax.experimental import pallas as pl


def kernel(X, edge_index, a, W1, b1, W2, b2, moment):
    raise NotImplementedError("write your pallas kernel here")


# speedup vs baseline: 3.2256x; 3.2256x over previous
import functools

import jax
import jax.numpy as jnp
from jax import lax
from jax.experimental import pallas as pl
from jax.experimental.pallas import tpu as pltpu
from jax.experimental.pallas import tpu_sc as plsc

N = 10000
NP = 10240
D = 128
NSC = 16
NC = 2
NW = NC * NSC
CH = 128
E = 320000
K = -(-E // (NW * CH))
EPAD = K * NW * CH
TRASH = NP - 1
RT = NP // NW
RS = NP // NSC
SUB = 64

_MESH = plsc.VectorSubcoreMesh(core_axis_name="c", subcore_axis_name="s")

f32 = jnp.float32


@functools.partial(
    pl.kernel,
    out_type=jax.ShapeDtypeStruct((NC, NP, 16), f32),
    mesh=_MESH,
    scratch_types=[
        pltpu.MemorySpace.VMEM_SHARED((NP, 16), f32),
        pltpu.VMEM((K, CH), jnp.int32),
        pltpu.VMEM((CH, 16), f32),
    ],
)
def _count(cols_hbm, ones_hbm, z16_hbm, degp, deg, colsv, onesv):
    c = lax.axis_index("c")
    s = lax.axis_index("s")
    wid = c * NSC + s
    pltpu.sync_copy(z16_hbm, deg.at[pl.ds(s * RS, RS)])
    pltpu.sync_copy(ones_hbm, onesv)
    pltpu.sync_copy(cols_hbm.at[wid], colsv)
    plsc.subcore_barrier()

    @pl.loop(0, K)
    def _(j):
        pltpu.sync_copy(onesv, deg.at[colsv.at[j]], add=True)

    plsc.subcore_barrier()
    pltpu.sync_copy(deg.at[pl.ds(s * RS, RS)], degp.at[c, pl.ds(s * RS, RS)])


def _norm_body(d0_ref, d1_ref, x_ref, dm_ref, di_ref, u_ref, v_ref):
    deg = d0_ref[...] + d1_ref[...]
    dm = lax.rsqrt(deg + 1.0)
    di = 1.0 / deg
    dm_ref[...] = dm
    di_ref[...] = di
    x = x_ref[...]
    u_ref[...] = x * dm[:, 0:1]
    v_ref[...] = x * di[:, 0:1]


def _norm(degp, Xp):
    blk16 = pl.BlockSpec((1024, 16), lambda i: (i, 0))
    blkD = pl.BlockSpec((1024, D), lambda i: (i, 0))
    return pl.pallas_call(
        _norm_body,
        grid=(NP // 1024,),
        in_specs=[blk16, blk16, blkD],
        out_specs=[blk16, blk16, blkD, blkD],
        out_shape=(
            jax.ShapeDtypeStruct((NP, 16), f32),
            jax.ShapeDtypeStruct((NP, 16), f32),
            jax.ShapeDtypeStruct((NP, D), f32),
            jax.ShapeDtypeStruct((NP, D), f32),
        ),
    )(degp[0], degp[1], Xp)


@functools.partial(
    pl.kernel,
    out_type=jax.ShapeDtypeStruct((NC, NP, D), f32),
    mesh=_MESH,
    scratch_types=[
        pltpu.MemorySpace.VMEM_SHARED((NP, D), f32),
        pltpu.VMEM((K, CH), jnp.int32),
        pltpu.VMEM((K, CH), jnp.int32),
        pltpu.VMEM((CH, D), f32),
    ],
)
def _spmm(u_hbm, cols_hbm, rows_hbm, z_hbm, p_out, acc, colsv, rowsv, gbuf):
    c = lax.axis_index("c")
    s = lax.axis_index("s")
    wid = c * NSC + s
    pltpu.sync_copy(z_hbm, acc.at[pl.ds(s * RS, RS)])
    pltpu.sync_copy(cols_hbm.at[wid], colsv)
    pltpu.sync_copy(rows_hbm.at[wid], rowsv)
    plsc.subcore_barrier()

    @pl.loop(0, K)
    def _(j):
        pltpu.sync_copy(u_hbm.at[colsv.at[j]], gbuf)
        pltpu.sync_copy(gbuf, acc.at[rowsv.at[j]], add=True)

    plsc.subcore_barrier()
    pltpu.sync_copy(acc.at[pl.ds(s * RS, RS)], p_out.at[c, pl.ds(s * RS, RS)])


@functools.partial(
    pl.kernel,
    out_type=(
        jax.ShapeDtypeStruct((NP, D), f32),
        jax.ShapeDtypeStruct((NP, D), f32),
    ),
    mesh=_MESH,
    scratch_types=[
        pltpu.VMEM((SUB, D), f32),
        pltpu.VMEM((SUB, D), f32),
        pltpu.VMEM((SUB, D), f32),
        pltpu.VMEM((SUB, 16), f32),
    ],
)
def _gcn_dense(p_hbm, uprev_hbm, dm_hbm, f_out, u_out, b0, b1, b2, dmv):
    c = lax.axis_index("c")
    s = lax.axis_index("s")
    base = (c * NSC + s) * RT
    for m in range(RT // SUB):
        r0 = base + m * SUB
        pltpu.sync_copy(p_hbm.at[0, pl.ds(r0, SUB)], b0)
        pltpu.sync_copy(p_hbm.at[1, pl.ds(r0, SUB)], b1)
        pltpu.sync_copy(uprev_hbm.at[pl.ds(r0, SUB)], b2)
        pltpu.sync_copy(dm_hbm.at[pl.ds(r0, SUB)], dmv)

        @pl.loop(0, SUB)
        def _(i):
            dm = dmv[i]
            for q in range(D // 16):
                sl = pl.ds(q * 16, 16)
                t = b0[i, sl] + b1[i, sl] + b2[i, sl]
                f = t * dm
                b0[i, sl] = f
                b1[i, sl] = f * dm

        pltpu.sync_copy(b0, f_out.at[pl.ds(r0, SUB)])
        pltpu.sync_copy(b1, u_out.at[pl.ds(r0, SUB)])


@functools.partial(
    pl.kernel,
    out_type=(
        jax.ShapeDtypeStruct((NP, D), f32),
        jax.ShapeDtypeStruct((NP, D), f32),
    ),
    mesh=_MESH,
    scratch_types=[
        pltpu.VMEM((SUB, D), f32),
        pltpu.VMEM((SUB, D), f32),
        pltpu.VMEM((SUB, D), f32),
        pltpu.VMEM((SUB, 16), f32),
    ],
)
def _scat_dense(p_hbm, fprev_hbm, di_hbm, fp_out, v_out, b0, b1, b2, div):
    c = lax.axis_index("c")
    s = lax.axis_index("s")
    base = (c * NSC + s) * RT
    for m in range(RT // SUB):
        r0 = base + m * SUB
        pltpu.sync_copy(p_hbm.at[0, pl.ds(r0, SUB)], b0)
        pltpu.sync_copy(p_hbm.at[1, pl.ds(r0, SUB)], b1)
        pltpu.sync_copy(fprev_hbm.at[pl.ds(r0, SUB)], b2)
        pltpu.sync_copy(di_hbm.at[pl.ds(r0, SUB)], div)

        @pl.loop(0, SUB)
        def _(i):
            di = div[i]
            for q in range(D // 16):
                sl = pl.ds(q * 16, 16)
                fp = 0.5 * b2[i, sl] + 0.5 * (b0[i, sl] + b1[i, sl])
                b0[i, sl] = fp
                b1[i, sl] = fp * di

        pltpu.sync_copy(b0, fp_out.at[pl.ds(r0, SUB)])
        pltpu.sync_copy(b1, v_out.at[pl.ds(r0, SUB)])


_BLK = 1000


def _lrelu(x):
    return jnp.where(x >= 0, x, 0.01 * x)


def _dot_t(lhs, rhs):
    return lax.dot_general(lhs, rhs, (((1,), (1,)), ((), ())),
                           precision=lax.Precision.HIGHEST,
                           preferred_element_type=f32)


def _tc_body(x_ref, g1_ref, g2_ref, g3_ref, f1_ref, f2_ref, f3_ref, f4_ref,
             a_ref, w1_ref, b1_ref, w2_ref, b2_ref, o_ref):
    x = x_ref[...]
    f1, f2, f3, f4 = f1_ref[...], f2_ref[...], f3_ref[...], f4_ref[...]
    hs = [
        _lrelu(g1_ref[...]),
        _lrelu(g2_ref[...]),
        _lrelu(g3_ref[...]),
        jnp.abs(f1 - f2),
        jnp.abs(f2 - f3),
        jnp.abs(f3 - f4),
    ]
    a1 = a_ref[:, :D]
    a2 = a_ref[:, D:]
    c0 = _dot_t(jnp.maximum(x, 0.0), a1)
    e = jnp.concatenate(
        [c0 + _dot_t(jnp.maximum(h, 0.0), a2) for h in hs], axis=1)
    mx = jnp.max(e, axis=1, keepdims=True)
    w = jnp.exp(e - mx)
    att = w / jnp.sum(w, axis=1, keepdims=True)
    hp = att[:, 0:1] * hs[0]
    for kk in range(1, 6):
        hp = hp + att[:, kk:kk + 1] * hs[kk]
    hp = hp * (1.0 / 6.0)
    o = _lrelu(_dot_t(hp, w1_ref[...]) + b1_ref[...])
    o = _lrelu(_dot_t(o, w2_ref[...]) + b2_ref[...])
    o_ref[...] = o


def _tc_tail(X, g1, g2, g3, fp1, fp2, fp3, fp4, a_r, W1, b1_r, W2, b2_r):
    big = pl.BlockSpec((_BLK, D), lambda i: (i, 0))
    full = lambda shp: pl.BlockSpec(shp, lambda i: tuple(0 for _ in shp))
    return pl.pallas_call(
        _tc_body,
        grid=(N // _BLK,),
        in_specs=[big] * 8 + [full((1, 2 * D)), full((D, D)), full((1, D)),
                              full((D, D)), full((1, D))],
        out_specs=big,
        out_shape=jax.ShapeDtypeStruct((N, D), f32),
    )(X, g1, g2, g3, fp1, fp2, fp3, fp4, a_r, W1, b1_r, W2, b2_r)


def kernel(X, edge_index, a, W1, b1, W2, b2, moment):
    rows = edge_index[0].astype(jnp.int32)
    cols = edge_index[1].astype(jnp.int32)
    pad = jnp.full((EPAD - E,), TRASH, jnp.int32)
    cols3 = jnp.concatenate([cols, pad]).reshape(NW, K, CH)
    rows3 = jnp.concatenate([rows, pad]).reshape(NW, K, CH)

    Xp = jnp.pad(X, ((0, NP - N), (0, 0)))
    ones16 = jnp.ones((CH, 16), f32)
    z16 = jnp.zeros((RS, 16), f32)
    zD = jnp.zeros((RS, D), f32)

    degp = _count(cols3, ones16, z16)
    dm16, di16, u, v = _norm(degp, Xp)

    gcns = []
    for _ in range(3):
        p = _spmm(u, cols3, rows3, zD)
        f, u = _gcn_dense(p, u, dm16)
        gcns.append(f)

    fps = []
    fprev = Xp
    for _ in range(4):
        p = _spmm(v, cols3, rows3, zD)
        fprev, v = _scat_dense(p, fprev, di16)
        fps.append(fprev)

    return _tc_tail(X, gcns[0], gcns[1], gcns[2],
                    fps[0], fps[1], fps[2], fps[3],
                    a.reshape(1, 2 * D), W1, b1.reshape(1, D),
                    W2, b2.reshape(1, D))
